# baseline (speedup 1.0000x reference)
import functools

import jax
import jax.numpy as jnp
from jax import lax
from jax.experimental import pallas as pl
from jax.experimental.pallas import tpu as pltpu
from jax.experimental.pallas import tpu_sc as plsc

N = 10000
D = 256
HH = 256
HC = 128
RB = 2000
K = 80
NC = 2
NS = 16
NL = 16
NPAD = 10240
PAKM = 16384


def _transform_body(x_ref, w_ref, b_ref, wa_ref, ba_ref,
                    xw_ref, a_ref, m_ref, msc):
    i = pl.program_id(0)
    xw = jnp.dot(x_ref[...], w_ref[...],
                 preferred_element_type=jnp.float32) + b_ref[...]
    xw_ref[...] = xw
    a = jnp.dot(xw, wa_ref[...],
                preferred_element_type=jnp.float32) + ba_ref[...]
    a_ref[...] = a
    bm = jnp.max(a)

    @pl.when(i == 0)
    def _():
        msc[0, 0] = bm

    @pl.when(i > 0)
    def _():
        msc[0, 0] = jnp.maximum(msc[0, 0], bm)

    m_ref[...] = jnp.full((1, 1), msc[0, 0], jnp.float32)


def _node_transform(x, w, b, wa, ba):
    return pl.pallas_call(
        _transform_body,
        grid=(N // RB,),
        in_specs=[
            pl.BlockSpec((RB, D), lambda i: (i, 0)),
            pl.BlockSpec((D, HH), lambda i: (0, 0)),
            pl.BlockSpec((1, HH), lambda i: (0, 0)),
            pl.BlockSpec((D, 1), lambda i: (0, 0)),
            pl.BlockSpec((1, 1), lambda i: (0, 0)),
        ],
        out_specs=[
            pl.BlockSpec((RB, HH), lambda i: (i, 0)),
            pl.BlockSpec((RB, 1), lambda i: (i, 0)),
            pl.BlockSpec((1, 1), lambda i: (0, 0)),
        ],
        out_shape=[
            jax.ShapeDtypeStruct((N, HH), jnp.float32),
            jax.ShapeDtypeStruct((N, 1), jnp.float32),
            jax.ShapeDtypeStruct((1, 1), jnp.float32),
        ],
        scratch_shapes=[pltpu.SMEM((1, 1), jnp.float32)],
    )(x, w, b.reshape(1, HH), wa, ba.reshape(1, 1))


def _weight_body(xw_ref, a_ref, m_ref, y_ref, w_ref):
    w = jnp.exp(a_ref[...] - m_ref[0, 0])
    w_ref[0, :, :] = w
    w_ref[1, :, :] = w
    y = xw_ref[...] * w
    y_ref[0, :, :] = y[:, :HC]
    y_ref[1, :, :] = y[:, HC:]


def _node_weight(xw, a, m):
    return pl.pallas_call(
        _weight_body,
        grid=(N // RB,),
        in_specs=[
            pl.BlockSpec((RB, HH), lambda i: (i, 0)),
            pl.BlockSpec((RB, 1), lambda i: (i, 0)),
            pl.BlockSpec((1, 1), lambda i: (0, 0)),
        ],
        out_specs=[
            pl.BlockSpec((2, RB, HC), lambda i: (0, i, 0)),
            pl.BlockSpec((2, RB, 1), lambda i: (0, i, 0)),
        ],
        out_shape=[
            jax.ShapeDtypeStruct((2, N, HC), jnp.float32),
            jax.ShapeDtypeStruct((2, N, 1), jnp.float32),
        ],
    )(xw, a, m)


def _sc_segment2(ycat1, wcat1, pak1, ycat2, wcat2, pak2, zero):
    nch = pak1.shape[1]
    assert pak1.shape == (NS, nch, K) and nch % 2 == 1
    half = (nch - 1) // 2
    stripe = NPAD // NS
    mesh = plsc.VectorSubcoreMesh(core_axis_name="c", subcore_axis_name="s")

    @functools.partial(
        pl.kernel,
        mesh=mesh,
        out_type=[
            jax.ShapeDtypeStruct((2 * NPAD, HC), jnp.float32),
            jax.ShapeDtypeStruct((2 * NPAD, HC), jnp.float32),
            jax.ShapeDtypeStruct((2 * NC * NS * NL,), jnp.float32),
        ],
        scratch_types=[
            pltpu.VMEM((nch, K), jnp.int32),
            pltpu.VMEM((K,), jnp.int32),
            pltpu.VMEM((K,), jnp.int32),
            pltpu.VMEM((K,), jnp.int32),
            pltpu.VMEM((K,), jnp.int32),
            pltpu.VMEM((K, HC), jnp.float32),
            pltpu.VMEM((K, HC), jnp.float32),
            pltpu.VMEM((K,), jnp.float32),
            pltpu.VMEM((K,), jnp.float32),
            pltpu.VMEM((NL,), jnp.float32),
            pltpu.VMEM_SHARED((NPAD, HC), jnp.float32),
            pltpu.SemaphoreType.DMA,
            pltpu.SemaphoreType.DMA,
            pltpu.SemaphoreType.DMA,
            pltpu.SemaphoreType.DMA,
        ],
    )
    def k(ycat1_hbm, w1_hbm, pak1_hbm, ycat2_hbm, w2_hbm, pak2_hbm,
          zero_hbm, s1_hbm, s2_hbm, z_hbm,
          pakA, idxa0, idxa1, dstb0, dstb1, rows0, rows1, wch0, wch1,
          zacc_v, acc_sh, semr0, semr1, semw0, semw1):
        c = lax.axis_index("c")
        s = lax.axis_index("s")
        wid = c * NS + s
        roff = c * N
        woff = c * NPAD

        bufs = ((idxa0, dstb0, rows0, wch0, semr0, semw0),
                (idxa1, dstb1, rows1, wch1, semr1, semw1))

        def run_phase(ycat_hbm, w_hbm, pak_hbm, s_hbm, zoff):
            pltpu.sync_copy(pak_hbm.at[s], pakA)
            zacc_v[...] = jnp.zeros((NL,), jnp.float32)
            plsc.subcore_barrier()

            def fire(j, p):
                idxa, dstb, rows, wch, semr, semw = bufs[p]
                for t in range(K // NL):
                    v = pakA[j, pl.ds(NL * t, NL)]
                    dstb[pl.ds(NL * t, NL)] = lax.shift_right_logical(v, 14)
                    idxa[pl.ds(NL * t, NL)] = (v & (PAKM - 1)) + roff
                pltpu.async_copy(ycat_hbm.at[idxa], rows, semr)
                pltpu.async_copy(w_hbm.at[idxa], wch, semw)

            def consume(j, p):
                idxa, dstb, rows, wch, semr, semw = bufs[p]
                pltpu.make_async_copy(ycat_hbm.at[idxa], rows, semr).wait()
                pltpu.make_async_copy(w_hbm.at[idxa], wch, semw).wait()
                pltpu.sync_copy(rows, acc_sh.at[dstb], add=True)
                zv = zacc_v[...]
                for t in range(K // NL):
                    zv = zv + wch[pl.ds(NL * t, NL)]
                zacc_v[...] = zv

            fire(0, 0)

            def body(i, carry):
                fire(2 * i + 1, 1)
                consume(2 * i, 0)
                fire(2 * i + 2, 0)
                consume(2 * i + 1, 1)
                return carry

            lax.fori_loop(0, half, body, 0)
            consume(nch - 1, 0)
            plsc.subcore_barrier()
            pltpu.sync_copy(acc_sh.at[pl.ds(s * stripe, stripe)],
                            s_hbm.at[pl.ds(woff + s * stripe, stripe)])
            pltpu.sync_copy(zacc_v, z_hbm.at[pl.ds(zoff + wid * NL, NL)])

        pltpu.sync_copy(zero_hbm.at[pl.ds(s * stripe, stripe)],
                        acc_sh.at[pl.ds(s * stripe, stripe)])
        run_phase(ycat1_hbm, w1_hbm, pak1_hbm, s1_hbm, 0)
        pltpu.sync_copy(zero_hbm.at[pl.ds(s * stripe, stripe)],
                        acc_sh.at[pl.ds(s * stripe, stripe)])
        run_phase(ycat2_hbm, w2_hbm, pak2_hbm, s2_hbm, NC * NS * NL)

    return k(ycat1, wcat1, pak1, ycat2, wcat2, pak2, zero)


def _update_body(x_ref, s0_ref, s1_ref, z_ref, wu_ref, bu_ref, o_ref):
    zinv = 2.0 / jnp.sum(z_ref[...])
    x = x_ref[...]
    acc = jnp.dot(x, wu_ref[0:D, :], preferred_element_type=jnp.float32)
    msum = jnp.dot(s0_ref[...], wu_ref[D:D + HC, :],
                   preferred_element_type=jnp.float32)
    msum += jnp.dot(s1_ref[...], wu_ref[D + HC:, :],
                    preferred_element_type=jnp.float32)
    o_ref[...] = jax.nn.relu(x + acc + msum * zinv + bu_ref[...])


def _node_update(x, s, z, wu, bu):
    s0 = lax.slice(s, (0, 0), (N, HC))
    s1 = lax.slice(s, (NPAD, 0), (NPAD + N, HC))
    nb = N // RB
    return pl.pallas_call(
        _update_body,
        grid=(nb,),
        in_specs=[
            pl.BlockSpec((RB, D), lambda i: (i, 0)),
            pl.BlockSpec((RB, HC), lambda i: (i, 0)),
            pl.BlockSpec((RB, HC), lambda i: (i, 0)),
            pl.BlockSpec((1, NC * NS * NL), lambda i: (0, 0)),
            pl.BlockSpec((2 * D, HH), lambda i: (0, 0)),
            pl.BlockSpec((1, HH), lambda i: (0, 0)),
        ],
        out_specs=pl.BlockSpec((RB, D), lambda i: (i, 0)),
        out_shape=jax.ShapeDtypeStruct((N, D), jnp.float32),
    )(x, s0, s1, z.reshape(1, -1), wu, bu.reshape(1, HH))


def kernel(place_features, transition_features, pre_edge_index, post_edge_index,
           W_ptm, b_ptm, W_tpm, b_tpm, W_pu, b_pu, W_tu, b_tu,
           W_pa, b_pa, W_ta, b_ta):
    E = pre_edge_index.shape[1]
    nch = E // (NS * K)
    pre = pre_edge_index.astype(jnp.int32)
    post = post_edge_index.astype(jnp.int32)
    pak_pre = (pre[0] + pre[1] * PAKM).reshape(NS, nch, K)
    pak_post = (post[0] + post[1] * PAKM).reshape(NS, nch, K)
    zero = jnp.zeros((NPAD, HC), jnp.float32)

    xw_p, a_p, m_p = _node_transform(place_features, W_ptm, b_ptm, W_ta, b_ta)
    y_p, w_p = _node_weight(xw_p, a_p, m_p)

    xw_t, a_t, m_t = _node_transform(transition_features, W_tpm, b_tpm,
                                     W_pa, b_pa)
    y_t, w_t = _node_weight(xw_t, a_t, m_t)

    s_p, s_t, z_pt = _sc_segment2(
        y_p.reshape(2 * N, HC), w_p.reshape(2 * N), pak_pre,
        y_t.reshape(2 * N, HC), w_t.reshape(2 * N), pak_post, zero)
    z_p = lax.slice(z_pt, (0,), (NC * NS * NL,))
    z_t = lax.slice(z_pt, (NC * NS * NL,), (2 * NC * NS * NL,))

    trans_out = _node_update(transition_features, s_p, z_p, W_tu, b_tu)
    place_out = _node_update(place_features, s_t, z_t, W_pu, b_pu)
    return (place_out, trans_out)


# speedup vs baseline: 1.0660x; 1.0660x over previous
import functools

import jax
import jax.numpy as jnp
from jax import lax
from jax.experimental import pallas as pl
from jax.experimental.pallas import tpu as pltpu
from jax.experimental.pallas import tpu_sc as plsc

N = 10000
D = 256
HH = 256
HC = 128
RB = 2000
K = 80
NC = 2
NS = 16
NL = 16
NPAD = 10240
PAKM = 16384


def _transform_body(x_ref, w_ref, b_ref, wa_ref, ba_ref,
                    xw_ref, a_ref, m_ref, msc):
    i = pl.program_id(0)
    xw = jnp.dot(x_ref[...], w_ref[...],
                 preferred_element_type=jnp.float32) + b_ref[...]
    xw_ref[...] = xw
    a = jnp.dot(xw, wa_ref[...],
                preferred_element_type=jnp.float32) + ba_ref[...]
    a_ref[...] = a
    bm = jnp.max(a)

    @pl.when(i == 0)
    def _():
        msc[0, 0] = bm

    @pl.when(i > 0)
    def _():
        msc[0, 0] = jnp.maximum(msc[0, 0], bm)

    m_ref[...] = jnp.full((1, 1), msc[0, 0], jnp.float32)


def _node_transform(x, w, b, wa, ba):
    return pl.pallas_call(
        _transform_body,
        grid=(N // RB,),
        in_specs=[
            pl.BlockSpec((RB, D), lambda i: (i, 0)),
            pl.BlockSpec((D, HH), lambda i: (0, 0)),
            pl.BlockSpec((1, HH), lambda i: (0, 0)),
            pl.BlockSpec((D, 1), lambda i: (0, 0)),
            pl.BlockSpec((1, 1), lambda i: (0, 0)),
        ],
        out_specs=[
            pl.BlockSpec((RB, HH), lambda i: (i, 0)),
            pl.BlockSpec((RB, 1), lambda i: (i, 0)),
            pl.BlockSpec((1, 1), lambda i: (0, 0)),
        ],
        out_shape=[
            jax.ShapeDtypeStruct((N, HH), jnp.float32),
            jax.ShapeDtypeStruct((N, 1), jnp.float32),
            jax.ShapeDtypeStruct((1, 1), jnp.float32),
        ],
        scratch_shapes=[pltpu.SMEM((1, 1), jnp.float32)],
    )(x, w, b.reshape(1, HH), wa, ba.reshape(1, 1))


def _weight_body(xw_ref, a_ref, m_ref, y_ref, w_ref):
    w = jnp.exp(a_ref[...] - m_ref[0, 0])
    w_ref[0, :, :] = w
    w_ref[1, :, :] = w
    y = xw_ref[...] * w
    y_ref[0, :, :] = y[:, :HC]
    y_ref[1, :, :] = y[:, HC:]


def _node_weight(xw, a, m):
    return pl.pallas_call(
        _weight_body,
        grid=(N // RB,),
        in_specs=[
            pl.BlockSpec((RB, HH), lambda i: (i, 0)),
            pl.BlockSpec((RB, 1), lambda i: (i, 0)),
            pl.BlockSpec((1, 1), lambda i: (0, 0)),
        ],
        out_specs=[
            pl.BlockSpec((2, RB, HC), lambda i: (0, i, 0)),
            pl.BlockSpec((2, RB, 1), lambda i: (0, i, 0)),
        ],
        out_shape=[
            jax.ShapeDtypeStruct((2, N, HC), jnp.float32),
            jax.ShapeDtypeStruct((2, N, 1), jnp.float32),
        ],
    )(xw, a, m)


def _sc_segment(ycat, wcat, pak3, zero):
    nch = pak3.shape[1]
    assert pak3.shape == (NS, nch, K) and nch % 2 == 1
    half = (nch - 1) // 2
    stripe = NPAD // NS
    mesh = plsc.VectorSubcoreMesh(core_axis_name="c", subcore_axis_name="s")

    @functools.partial(
        pl.kernel,
        mesh=mesh,
        out_type=[
            jax.ShapeDtypeStruct((2 * NPAD, HC), jnp.float32),
            jax.ShapeDtypeStruct((NC * NS * NL,), jnp.float32),
        ],
        scratch_types=[
            pltpu.VMEM((nch, K), jnp.int32),
            pltpu.VMEM((K,), jnp.int32),
            pltpu.VMEM((K,), jnp.int32),
            pltpu.VMEM((K,), jnp.int32),
            pltpu.VMEM((K,), jnp.int32),
            pltpu.VMEM((K, HC), jnp.float32),
            pltpu.VMEM((K, HC), jnp.float32),
            pltpu.VMEM((K,), jnp.float32),
            pltpu.VMEM((K,), jnp.float32),
            pltpu.VMEM((NL,), jnp.float32),
            pltpu.VMEM_SHARED((NPAD, HC), jnp.float32),
            pltpu.SemaphoreType.DMA,
            pltpu.SemaphoreType.DMA,
            pltpu.SemaphoreType.DMA,
            pltpu.SemaphoreType.DMA,
            pltpu.SemaphoreType.DMA,
            pltpu.SemaphoreType.DMA,
        ],
    )
    def k(ycat_hbm, w_hbm, pak_hbm, zero_hbm, s_hbm, z_hbm,
          pakA, idxa0, idxa1, dstb0, dstb1, rows0, rows1, wch0, wch1,
          zacc_v, acc_sh, semr0, semr1, semw0, semw1, sems0, sems1):
        c = lax.axis_index("c")
        s = lax.axis_index("s")
        wid = c * NS + s
        roff = c * N
        woff = c * NPAD
        pltpu.sync_copy(zero_hbm.at[pl.ds(s * stripe, stripe)],
                        acc_sh.at[pl.ds(s * stripe, stripe)])
        pltpu.sync_copy(pak_hbm.at[s], pakA)
        zacc_v[...] = jnp.zeros((NL,), jnp.float32)
        plsc.subcore_barrier()

        bufs = ((idxa0, dstb0, rows0, wch0, semr0, semw0, sems0),
                (idxa1, dstb1, rows1, wch1, semr1, semw1, sems1))

        def gfire(j, p):
            idxa, dstb, rows, wch, semr, semw, sems = bufs[p]
            for t in range(K // NL):
                v = pakA[j, pl.ds(NL * t, NL)]
                dstb[pl.ds(NL * t, NL)] = lax.shift_right_logical(v, 14)
                idxa[pl.ds(NL * t, NL)] = (v & (PAKM - 1)) + roff
            pltpu.async_copy(ycat_hbm.at[idxa], rows, semr)
            pltpu.async_copy(w_hbm.at[idxa], wch, semw)

        def sfire(j, p):
            idxa, dstb, rows, wch, semr, semw, sems = bufs[p]
            pltpu.make_async_copy(ycat_hbm.at[idxa], rows, semr).wait()
            pltpu.make_async_copy(w_hbm.at[idxa], wch, semw).wait()
            pltpu.async_copy(rows, acc_sh.at[dstb], sems, add=True)
            zv = zacc_v[...]
            for t in range(K // NL):
                zv = zv + wch[pl.ds(NL * t, NL)]
            zacc_v[...] = zv

        def swait(j, p):
            idxa, dstb, rows, wch, semr, semw, sems = bufs[p]
            pltpu.make_async_copy(rows, acc_sh.at[dstb], sems).wait()

        gfire(0, 0)
        gfire(1, 1)
        sfire(0, 0)

        def body(i, carry):
            swait(2 * i - 2, 0)
            gfire(2 * i, 0)
            sfire(2 * i - 1, 1)
            swait(2 * i - 1, 1)
            gfire(2 * i + 1, 1)
            sfire(2 * i, 0)
            return carry

        lax.fori_loop(1, half, body, 0)
        sfire(nch - 2, 1)
        swait(nch - 2, 1)
        swait(nch - 3, 0)
        gfire(nch - 1, 0)
        sfire(nch - 1, 0)
        swait(nch - 1, 0)
        plsc.subcore_barrier()
        pltpu.sync_copy(acc_sh.at[pl.ds(s * stripe, stripe)],
                        s_hbm.at[pl.ds(woff + s * stripe, stripe)])
        pltpu.sync_copy(zacc_v, z_hbm.at[pl.ds(wid * NL, NL)])

    return k(ycat, wcat, pak3, zero)


def _update_body(x_ref, s0_ref, s1_ref, z_ref, wu_ref, bu_ref, o_ref):
    zinv = 2.0 / jnp.sum(z_ref[...])
    x = x_ref[...]
    acc = jnp.dot(x, wu_ref[0:D, :], preferred_element_type=jnp.float32)
    msum = jnp.dot(s0_ref[...], wu_ref[D:D + HC, :],
                   preferred_element_type=jnp.float32)
    msum += jnp.dot(s1_ref[...], wu_ref[D + HC:, :],
                    preferred_element_type=jnp.float32)
    o_ref[...] = jax.nn.relu(x + acc + msum * zinv + bu_ref[...])


def _node_update(x, s, z, wu, bu):
    s0 = lax.slice(s, (0, 0), (N, HC))
    s1 = lax.slice(s, (NPAD, 0), (NPAD + N, HC))
    nb = N // RB
    return pl.pallas_call(
        _update_body,
        grid=(nb,),
        in_specs=[
            pl.BlockSpec((RB, D), lambda i: (i, 0)),
            pl.BlockSpec((RB, HC), lambda i: (i, 0)),
            pl.BlockSpec((RB, HC), lambda i: (i, 0)),
            pl.BlockSpec((1, NC * NS * NL), lambda i: (0, 0)),
            pl.BlockSpec((2 * D, HH), lambda i: (0, 0)),
            pl.BlockSpec((1, HH), lambda i: (0, 0)),
        ],
        out_specs=pl.BlockSpec((RB, D), lambda i: (i, 0)),
        out_shape=jax.ShapeDtypeStruct((N, D), jnp.float32),
    )(x, s0, s1, z.reshape(1, -1), wu, bu.reshape(1, HH))


def kernel(place_features, transition_features, pre_edge_index, post_edge_index,
           W_ptm, b_ptm, W_tpm, b_tpm, W_pu, b_pu, W_tu, b_tu,
           W_pa, b_pa, W_ta, b_ta):
    E = pre_edge_index.shape[1]
    nch = E // (NS * K)
    pre = pre_edge_index.astype(jnp.int32)
    post = post_edge_index.astype(jnp.int32)
    pak_pre = (pre[0] + pre[1] * PAKM).reshape(NS, nch, K)
    pak_post = (post[0] + post[1] * PAKM).reshape(NS, nch, K)
    zero = jnp.zeros((NPAD, HC), jnp.float32)

    xw_p, a_p, m_p = _node_transform(place_features, W_ptm, b_ptm, W_ta, b_ta)
    y_p, w_p = _node_weight(xw_p, a_p, m_p)
    s_p, z_p = _sc_segment(y_p.reshape(2 * N, HC), w_p.reshape(2 * N),
                           pak_pre, zero)

    xw_t, a_t, m_t = _node_transform(transition_features, W_tpm, b_tpm,
                                     W_pa, b_pa)
    y_t, w_t = _node_weight(xw_t, a_t, m_t)
    s_t, z_t = _sc_segment(y_t.reshape(2 * N, HC), w_t.reshape(2 * N),
                           pak_post, zero)

    trans_out = _node_update(transition_features, s_p, z_p, W_tu, b_tu)
    place_out = _node_update(place_features, s_t, z_t, W_pu, b_pu)
    return (place_out, trans_out)


# speedup vs baseline: 1.1686x; 1.0963x over previous
import functools

import jax
import jax.numpy as jnp
from jax import lax
from jax.experimental import pallas as pl
from jax.experimental.pallas import tpu as pltpu
from jax.experimental.pallas import tpu_sc as plsc

N = 10000
D = 256
HH = 256
HC = 128
RB = 2000
K = 80
NC = 2
NS = 16
NL = 16
NPAD = 10240
PAKM = 16384


def _transform_body(x_ref, w_ref, b_ref, wa_ref, ba_ref,
                    xw_ref, a_ref, m_ref, msc):
    i = pl.program_id(0)
    xw = jnp.dot(x_ref[...], w_ref[...],
                 preferred_element_type=jnp.float32) + b_ref[...]
    xw_ref[...] = xw
    a = jnp.dot(xw, wa_ref[...],
                preferred_element_type=jnp.float32) + ba_ref[...]
    a_ref[...] = a
    bm = jnp.max(a)

    @pl.when(i == 0)
    def _():
        msc[0, 0] = bm

    @pl.when(i > 0)
    def _():
        msc[0, 0] = jnp.maximum(msc[0, 0], bm)

    m_ref[...] = jnp.full((1, 1), msc[0, 0], jnp.float32)


def _node_transform(x, w, b, wa, ba):
    return pl.pallas_call(
        _transform_body,
        grid=(N // RB,),
        in_specs=[
            pl.BlockSpec((RB, D), lambda i: (i, 0)),
            pl.BlockSpec((D, HH), lambda i: (0, 0)),
            pl.BlockSpec((1, HH), lambda i: (0, 0)),
            pl.BlockSpec((D, 1), lambda i: (0, 0)),
            pl.BlockSpec((1, 1), lambda i: (0, 0)),
        ],
        out_specs=[
            pl.BlockSpec((RB, HH), lambda i: (i, 0)),
            pl.BlockSpec((RB, 1), lambda i: (i, 0)),
            pl.BlockSpec((1, 1), lambda i: (0, 0)),
        ],
        out_shape=[
            jax.ShapeDtypeStruct((N, HH), jnp.float32),
            jax.ShapeDtypeStruct((N, 1), jnp.float32),
            jax.ShapeDtypeStruct((1, 1), jnp.float32),
        ],
        scratch_shapes=[pltpu.SMEM((1, 1), jnp.float32)],
    )(x, w, b.reshape(1, HH), wa, ba.reshape(1, 1))


def _weight_body(xw_ref, a_ref, m_ref, y_ref, w_ref):
    w = jnp.exp(a_ref[...] - m_ref[0, 0])
    w_ref[0, :, :] = w
    w_ref[1, :, :] = w
    y = xw_ref[...] * w
    y_ref[0, :, :] = y[:, :HC]
    y_ref[1, :, :] = y[:, HC:]


def _node_weight(xw, a, m):
    return pl.pallas_call(
        _weight_body,
        grid=(N // RB,),
        in_specs=[
            pl.BlockSpec((RB, HH), lambda i: (i, 0)),
            pl.BlockSpec((RB, 1), lambda i: (i, 0)),
            pl.BlockSpec((1, 1), lambda i: (0, 0)),
        ],
        out_specs=[
            pl.BlockSpec((2, RB, HC), lambda i: (0, i, 0)),
            pl.BlockSpec((2, RB, 1), lambda i: (0, i, 0)),
        ],
        out_shape=[
            jax.ShapeDtypeStruct((2, N, HC), jnp.float32),
            jax.ShapeDtypeStruct((2, N, 1), jnp.float32),
        ],
    )(xw, a, m)


def _sc_segment(ycat, wcat, pak3, zero):
    P, G = 3, 1
    nch = pak3.shape[1]
    assert pak3.shape == (NS, nch, K) and nch >= 2 * P + G
    stripe = NPAD // NS
    mesh = plsc.VectorSubcoreMesh(core_axis_name="c", subcore_axis_name="s")

    @functools.partial(
        pl.kernel,
        mesh=mesh,
        out_type=[
            jax.ShapeDtypeStruct((2 * NPAD, HC), jnp.float32),
            jax.ShapeDtypeStruct((NC * NS * NL,), jnp.float32),
        ],
        scratch_types=[
            pltpu.VMEM((nch, K), jnp.int32),
            *([pltpu.VMEM((K,), jnp.int32)] * (2 * P)),
            *([pltpu.VMEM((K, HC), jnp.float32)] * P),
            *([pltpu.VMEM((K,), jnp.float32)] * P),
            pltpu.VMEM((NL,), jnp.float32),
            pltpu.VMEM_SHARED((NPAD, HC), jnp.float32),
            *([pltpu.SemaphoreType.DMA] * (3 * P)),
        ],
    )
    def k(ycat_hbm, w_hbm, pak_hbm, zero_hbm, s_hbm, z_hbm,
          pakA, *rest):
        idxas = rest[0:P]
        dstbs = rest[P:2 * P]
        rowss = rest[2 * P:3 * P]
        wchs = rest[3 * P:4 * P]
        zacc_v = rest[4 * P]
        acc_sh = rest[4 * P + 1]
        semrs = rest[4 * P + 2:5 * P + 2]
        semws = rest[5 * P + 2:6 * P + 2]
        semss = rest[6 * P + 2:7 * P + 2]
        c = lax.axis_index("c")
        s = lax.axis_index("s")
        wid = c * NS + s
        roff = c * N
        woff = c * NPAD
        pltpu.sync_copy(zero_hbm.at[pl.ds(s * stripe, stripe)],
                        acc_sh.at[pl.ds(s * stripe, stripe)])
        pltpu.sync_copy(pak_hbm.at[s], pakA)
        zacc_v[...] = jnp.zeros((NL,), jnp.float32)
        plsc.subcore_barrier()

        def gfire(j, p):
            idxa, dstb, rows, wch = idxas[p], dstbs[p], rowss[p], wchs[p]
            for t in range(K // NL):
                v = pakA[j, pl.ds(NL * t, NL)]
                dstb[pl.ds(NL * t, NL)] = lax.shift_right_logical(v, 14)
                idxa[pl.ds(NL * t, NL)] = (v & (PAKM - 1)) + roff
            pltpu.async_copy(ycat_hbm.at[idxa], rows, semrs[p])
            pltpu.async_copy(w_hbm.at[idxa], wch, semws[p])

        def sfire(j, p):
            idxa, dstb, rows, wch = idxas[p], dstbs[p], rowss[p], wchs[p]
            pltpu.make_async_copy(ycat_hbm.at[idxa], rows, semrs[p]).wait()
            pltpu.make_async_copy(w_hbm.at[idxa], wch, semws[p]).wait()
            pltpu.async_copy(rows, acc_sh.at[dstb], semss[p], add=True)
            zv = zacc_v[...]
            for t in range(K // NL):
                zv = zv + wch[pl.ds(NL * t, NL)]
            zacc_v[...] = zv

        def swait(j, p):
            pltpu.make_async_copy(rowss[p], acc_sh.at[dstbs[p]],
                                  semss[p]).wait()

        def slot(j, r):
            jw = j - (P - G)
            if isinstance(j, int):
                if jw >= 0:
                    swait(jw, jw % P)
                if j + G < nch:
                    gfire(j + G, (j + G) % P)
            else:
                swait(jw, (r - (P - G)) % P)
                gfire(j + G, (r + G) % P)
            sfire(j, r)

        for j in range(G):
            gfire(j, j)
        for j in range(P):
            slot(j, j)

        def body(i, carry):
            for r in range(P):
                slot(i * P + r, r)
            return carry

        F = (nch - G) // P
        lax.fori_loop(1, F, body, 0)
        for j in range(F * P, nch):
            slot(j, j % P)
        for j in range(nch - (P - G), nch):
            swait(j, j % P)
        plsc.subcore_barrier()
        pltpu.sync_copy(acc_sh.at[pl.ds(s * stripe, stripe)],
                        s_hbm.at[pl.ds(woff + s * stripe, stripe)])
        pltpu.sync_copy(zacc_v, z_hbm.at[pl.ds(wid * NL, NL)])

    return k(ycat, wcat, pak3, zero)


def _update_body(x_ref, s0_ref, s1_ref, z_ref, wu_ref, bu_ref, o_ref):
    zinv = 2.0 / jnp.sum(z_ref[...])
    x = x_ref[...]
    acc = jnp.dot(x, wu_ref[0:D, :], preferred_element_type=jnp.float32)
    msum = jnp.dot(s0_ref[...], wu_ref[D:D + HC, :],
                   preferred_element_type=jnp.float32)
    msum += jnp.dot(s1_ref[...], wu_ref[D + HC:, :],
                    preferred_element_type=jnp.float32)
    o_ref[...] = jax.nn.relu(x + acc + msum * zinv + bu_ref[...])


def _node_update(x, s, z, wu, bu):
    s0 = lax.slice(s, (0, 0), (N, HC))
    s1 = lax.slice(s, (NPAD, 0), (NPAD + N, HC))
    nb = N // RB
    return pl.pallas_call(
        _update_body,
        grid=(nb,),
        in_specs=[
            pl.BlockSpec((RB, D), lambda i: (i, 0)),
            pl.BlockSpec((RB, HC), lambda i: (i, 0)),
            pl.BlockSpec((RB, HC), lambda i: (i, 0)),
            pl.BlockSpec((1, NC * NS * NL), lambda i: (0, 0)),
            pl.BlockSpec((2 * D, HH), lambda i: (0, 0)),
            pl.BlockSpec((1, HH), lambda i: (0, 0)),
        ],
        out_specs=pl.BlockSpec((RB, D), lambda i: (i, 0)),
        out_shape=jax.ShapeDtypeStruct((N, D), jnp.float32),
    )(x, s0, s1, z.reshape(1, -1), wu, bu.reshape(1, HH))


def kernel(place_features, transition_features, pre_edge_index, post_edge_index,
           W_ptm, b_ptm, W_tpm, b_tpm, W_pu, b_pu, W_tu, b_tu,
           W_pa, b_pa, W_ta, b_ta):
    E = pre_edge_index.shape[1]
    nch = E // (NS * K)
    pre = pre_edge_index.astype(jnp.int32)
    post = post_edge_index.astype(jnp.int32)
    pak_pre = (pre[0] + pre[1] * PAKM).reshape(NS, nch, K)
    pak_post = (post[0] + post[1] * PAKM).reshape(NS, nch, K)
    zero = jnp.zeros((NPAD, HC), jnp.float32)

    xw_p, a_p, m_p = _node_transform(place_features, W_ptm, b_ptm, W_ta, b_ta)
    y_p, w_p = _node_weight(xw_p, a_p, m_p)
    s_p, z_p = _sc_segment(y_p.reshape(2 * N, HC), w_p.reshape(2 * N),
                           pak_pre, zero)

    xw_t, a_t, m_t = _node_transform(transition_features, W_tpm, b_tpm,
                                     W_pa, b_pa)
    y_t, w_t = _node_weight(xw_t, a_t, m_t)
    s_t, z_t = _sc_segment(y_t.reshape(2 * N, HC), w_t.reshape(2 * N),
                           pak_post, zero)

    trans_out = _node_update(transition_features, s_p, z_p, W_tu, b_tu)
    place_out = _node_update(place_features, s_t, z_t, W_pu, b_pu)
    return (place_out, trans_out)


# speedup vs baseline: 1.1924x; 1.0204x over previous
import functools

import jax
import jax.numpy as jnp
from jax import lax
from jax.experimental import pallas as pl
from jax.experimental.pallas import tpu as pltpu
from jax.experimental.pallas import tpu_sc as plsc

N = 10000
D = 256
HH = 256
HC = 128
RB = 2000
K = 80
NC = 2
NS = 16
NL = 16
NPAD = 10240
PAKM = 16384


def _transform_body(x_ref, w_ref, b_ref, wa_ref, ba_ref,
                    xw_ref, a_ref, m_ref, msc):
    i = pl.program_id(0)
    xw = jnp.dot(x_ref[...], w_ref[...],
                 preferred_element_type=jnp.float32) + b_ref[...]
    xw_ref[...] = xw
    a = jnp.dot(xw, wa_ref[...],
                preferred_element_type=jnp.float32) + ba_ref[...]
    a_ref[...] = a
    bm = jnp.max(a)

    @pl.when(i == 0)
    def _():
        msc[0, 0] = bm

    @pl.when(i > 0)
    def _():
        msc[0, 0] = jnp.maximum(msc[0, 0], bm)

    m_ref[...] = jnp.full((1, 1), msc[0, 0], jnp.float32)


def _node_transform(x, w, b, wa, ba):
    return pl.pallas_call(
        _transform_body,
        grid=(N // RB,),
        in_specs=[
            pl.BlockSpec((RB, D), lambda i: (i, 0)),
            pl.BlockSpec((D, HH), lambda i: (0, 0)),
            pl.BlockSpec((1, HH), lambda i: (0, 0)),
            pl.BlockSpec((D, 1), lambda i: (0, 0)),
            pl.BlockSpec((1, 1), lambda i: (0, 0)),
        ],
        out_specs=[
            pl.BlockSpec((RB, HH), lambda i: (i, 0)),
            pl.BlockSpec((RB, 1), lambda i: (i, 0)),
            pl.BlockSpec((1, 1), lambda i: (0, 0)),
        ],
        out_shape=[
            jax.ShapeDtypeStruct((N, HH), jnp.float32),
            jax.ShapeDtypeStruct((N, 1), jnp.float32),
            jax.ShapeDtypeStruct((1, 1), jnp.float32),
        ],
        scratch_shapes=[pltpu.SMEM((1, 1), jnp.float32)],
    )(x, w, b.reshape(1, HH), wa, ba.reshape(1, 1))


def _weight_body(xw_ref, a_ref, m_ref, y_ref, w_ref):
    w = jnp.exp(a_ref[...] - m_ref[0, 0])
    w_ref[0, :, :] = w
    w_ref[1, :, :] = w
    y = xw_ref[...] * w
    y_ref[0, :, :] = y[:, :HC]
    y_ref[1, :, :] = y[:, HC:]


def _node_weight(xw, a, m):
    return pl.pallas_call(
        _weight_body,
        grid=(N // RB,),
        in_specs=[
            pl.BlockSpec((RB, HH), lambda i: (i, 0)),
            pl.BlockSpec((RB, 1), lambda i: (i, 0)),
            pl.BlockSpec((1, 1), lambda i: (0, 0)),
        ],
        out_specs=[
            pl.BlockSpec((2, RB, HC), lambda i: (0, i, 0)),
            pl.BlockSpec((2, RB, 1), lambda i: (0, i, 0)),
        ],
        out_shape=[
            jax.ShapeDtypeStruct((2, N, HC), jnp.float32),
            jax.ShapeDtypeStruct((2, N, 1), jnp.float32),
        ],
    )(xw, a, m)


def _sc_segment(ycat, wcat, pak3, zero):
    P, G = 3, 2
    nch = pak3.shape[1]
    assert pak3.shape == (NS, nch, K) and nch >= 2 * P + G
    stripe = NPAD // NS
    mesh = plsc.VectorSubcoreMesh(core_axis_name="c", subcore_axis_name="s")

    @functools.partial(
        pl.kernel,
        mesh=mesh,
        out_type=[
            jax.ShapeDtypeStruct((2 * NPAD, HC), jnp.float32),
            jax.ShapeDtypeStruct((NC * NS * NL,), jnp.float32),
        ],
        scratch_types=[
            pltpu.VMEM((nch, K), jnp.int32),
            *([pltpu.VMEM((K,), jnp.int32)] * (2 * P)),
            *([pltpu.VMEM((K, HC), jnp.float32)] * P),
            *([pltpu.VMEM((K,), jnp.float32)] * P),
            pltpu.VMEM((NL,), jnp.float32),
            pltpu.VMEM_SHARED((NPAD, HC), jnp.float32),
            *([pltpu.SemaphoreType.DMA] * (3 * P)),
        ],
    )
    def k(ycat_hbm, w_hbm, pak_hbm, zero_hbm, s_hbm, z_hbm,
          pakA, *rest):
        idxas = rest[0:P]
        dstbs = rest[P:2 * P]
        rowss = rest[2 * P:3 * P]
        wchs = rest[3 * P:4 * P]
        zacc_v = rest[4 * P]
        acc_sh = rest[4 * P + 1]
        semrs = rest[4 * P + 2:5 * P + 2]
        semws = rest[5 * P + 2:6 * P + 2]
        semss = rest[6 * P + 2:7 * P + 2]
        c = lax.axis_index("c")
        s = lax.axis_index("s")
        wid = c * NS + s
        roff = c * N
        woff = c * NPAD
        pltpu.sync_copy(zero_hbm.at[pl.ds(s * stripe, stripe)],
                        acc_sh.at[pl.ds(s * stripe, stripe)])
        pltpu.sync_copy(pak_hbm.at[s], pakA)
        zacc_v[...] = jnp.zeros((NL,), jnp.float32)
        plsc.subcore_barrier()

        def gfire(j, p):
            idxa, dstb, rows, wch = idxas[p], dstbs[p], rowss[p], wchs[p]
            for t in range(K // NL):
                v = pakA[j, pl.ds(NL * t, NL)]
                dstb[pl.ds(NL * t, NL)] = lax.shift_right_logical(v, 14)
                idxa[pl.ds(NL * t, NL)] = (v & (PAKM - 1)) + roff
            pltpu.async_copy(ycat_hbm.at[idxa], rows, semrs[p])
            pltpu.async_copy(w_hbm.at[idxa], wch, semws[p])

        def sfire(j, p):
            idxa, dstb, rows, wch = idxas[p], dstbs[p], rowss[p], wchs[p]
            pltpu.make_async_copy(ycat_hbm.at[idxa], rows, semrs[p]).wait()
            pltpu.make_async_copy(w_hbm.at[idxa], wch, semws[p]).wait()
            pltpu.async_copy(rows, acc_sh.at[dstb], semss[p], add=True)
            zv = zacc_v[...]
            for t in range(K // NL):
                zv = zv + wch[pl.ds(NL * t, NL)]
            zacc_v[...] = zv

        def swait(j, p):
            pltpu.make_async_copy(rowss[p], acc_sh.at[dstbs[p]],
                                  semss[p]).wait()

        def slot(j, r):
            jw = j - (P - G)
            if isinstance(j, int):
                if jw >= 0:
                    swait(jw, jw % P)
                if j + G < nch:
                    gfire(j + G, (j + G) % P)
            else:
                swait(jw, (r - (P - G)) % P)
                gfire(j + G, (r + G) % P)
            sfire(j, r)

        for j in range(G):
            gfire(j, j)
        for j in range(P):
            slot(j, j)

        def body(i, carry):
            for r in range(P):
                slot(i * P + r, r)
            return carry

        F = (nch - G) // P
        lax.fori_loop(1, F, body, 0)
        for j in range(F * P, nch):
            slot(j, j % P)
        for j in range(nch - (P - G), nch):
            swait(j, j % P)
        plsc.subcore_barrier()
        pltpu.sync_copy(acc_sh.at[pl.ds(s * stripe, stripe)],
                        s_hbm.at[pl.ds(woff + s * stripe, stripe)])
        pltpu.sync_copy(zacc_v, z_hbm.at[pl.ds(wid * NL, NL)])

    return k(ycat, wcat, pak3, zero)


def _update_body(x_ref, s0_ref, s1_ref, z_ref, wu_ref, bu_ref, o_ref):
    zinv = 2.0 / jnp.sum(z_ref[...])
    x = x_ref[...]
    acc = jnp.dot(x, wu_ref[0:D, :], preferred_element_type=jnp.float32)
    msum = jnp.dot(s0_ref[...], wu_ref[D:D + HC, :],
                   preferred_element_type=jnp.float32)
    msum += jnp.dot(s1_ref[...], wu_ref[D + HC:, :],
                    preferred_element_type=jnp.float32)
    o_ref[...] = jax.nn.relu(x + acc + msum * zinv + bu_ref[...])


def _node_update(x, s, z, wu, bu):
    s0 = lax.slice(s, (0, 0), (N, HC))
    s1 = lax.slice(s, (NPAD, 0), (NPAD + N, HC))
    nb = N // RB
    return pl.pallas_call(
        _update_body,
        grid=(nb,),
        in_specs=[
            pl.BlockSpec((RB, D), lambda i: (i, 0)),
            pl.BlockSpec((RB, HC), lambda i: (i, 0)),
            pl.BlockSpec((RB, HC), lambda i: (i, 0)),
            pl.BlockSpec((1, NC * NS * NL), lambda i: (0, 0)),
            pl.BlockSpec((2 * D, HH), lambda i: (0, 0)),
            pl.BlockSpec((1, HH), lambda i: (0, 0)),
        ],
        out_specs=pl.BlockSpec((RB, D), lambda i: (i, 0)),
        out_shape=jax.ShapeDtypeStruct((N, D), jnp.float32),
    )(x, s0, s1, z.reshape(1, -1), wu, bu.reshape(1, HH))


def kernel(place_features, transition_features, pre_edge_index, post_edge_index,
           W_ptm, b_ptm, W_tpm, b_tpm, W_pu, b_pu, W_tu, b_tu,
           W_pa, b_pa, W_ta, b_ta):
    E = pre_edge_index.shape[1]
    nch = E // (NS * K)
    pre = pre_edge_index.astype(jnp.int32)
    post = post_edge_index.astype(jnp.int32)
    pak_pre = (pre[0] + pre[1] * PAKM).reshape(NS, nch, K)
    pak_post = (post[0] + post[1] * PAKM).reshape(NS, nch, K)
    zero = jnp.zeros((NPAD, HC), jnp.float32)

    xw_p, a_p, m_p = _node_transform(place_features, W_ptm, b_ptm, W_ta, b_ta)
    y_p, w_p = _node_weight(xw_p, a_p, m_p)
    s_p, z_p = _sc_segment(y_p.reshape(2 * N, HC), w_p.reshape(2 * N),
                           pak_pre, zero)

    xw_t, a_t, m_t = _node_transform(transition_features, W_tpm, b_tpm,
                                     W_pa, b_pa)
    y_t, w_t = _node_weight(xw_t, a_t, m_t)
    s_t, z_t = _sc_segment(y_t.reshape(2 * N, HC), w_t.reshape(2 * N),
                           pak_post, zero)

    trans_out = _node_update(transition_features, s_p, z_p, W_tu, b_tu)
    place_out = _node_update(place_features, s_t, z_t, W_pu, b_pu)
    return (place_out, trans_out)
